# baseline (speedup 1.0000x reference)
import functools

import jax
import jax.numpy as jnp
from jax import lax
from jax.experimental import pallas as pl
from jax.experimental.pallas import tpu as pltpu
from jax.experimental.pallas import tpu_sc as plsc

N = 10000
E = 320000

_NC = 2
_NS = 16
_NW = _NC * _NS
_CH = 128
_CPT = 79
_EPT = _CPT * _CH
_EP = _NW * _EPT
_GRP = 5
_NPAD = 10240
_RPT = _NPAD // _NS

_W = 8


def _sc_body(edges_hbm, pidx_hbm, zeros_hbm, vrep_hbm,
             acc_hbm, eout_hbm,
             ebuf, pidx, sb, rb, vbuf, obuf, tbuf, acc_s, acc_r, stage, sem):
    cid = lax.axis_index("c")
    sid = lax.axis_index("s")
    wid = cid * _NS + sid
    pltpu.sync_copy(zeros_hbm, stage)
    pltpu.sync_copy(stage, acc_s.at[pl.ds(sid * _RPT, _RPT)])
    pltpu.sync_copy(stage, acc_r.at[pl.ds(sid * _RPT, _RPT)])
    plsc.subcore_barrier()
    pltpu.sync_copy(edges_hbm.at[wid], ebuf)
    pltpu.sync_copy(pidx_hbm.at[pl.ds(wid * _CPT, _CPT)], pidx)
    pltpu.sync_copy(vrep_hbm, vbuf)

    lane = lax.iota(jnp.int32, 16)

    def do_group(j0, nch):
        for t in range(nch):
            for c in range(_CH // 16):
                p = pidx[j0 + t, pl.ds(c * 16, 16)]
                sb[t, pl.ds(c * 16, 16)] = lax.bitwise_and(p, 0xFFFF)
                rb[t, pl.ds(c * 16, 16)] = lax.shift_right_logical(p, 16)
        descs = []
        for t in range(nch):
            src = ebuf.at[pl.ds((j0 + t) * _CH, _CH)]
            descs.append(
                pltpu.async_copy(src, acc_s.at[sb.at[t]], sem, add=True))
            descs.append(
                pltpu.async_copy(src, acc_r.at[rb.at[t]], sem, add=True))

        @pl.loop(0, nch * _CH // 16)
        def _dec(k):
            base = j0 * _CH + k * 16
            rows = base + lane
            r16 = vbuf[4]
            for f in range(4):
                cols = jnp.full((16,), f, jnp.int32)
                r16 = r16 + plsc.load_gather(ebuf, [rows, cols]) * vbuf[f]
            obuf[pl.ds(base, 16)] = r16

        for d in descs:
            d.wait()

    @pl.loop(0, 15)
    def _group(g):
        do_group(g * _GRP, _GRP)

    do_group(15 * _GRP, _CPT - 15 * _GRP)

    pltpu.sync_copy(obuf, eout_hbm.at[pl.ds(wid * _EPT, _EPT)])
    plsc.subcore_barrier()
    for t, acc in ((0, acc_s), (1, acc_r)):
        pltpu.sync_copy(acc.at[pl.ds(sid * _RPT, _RPT)], stage)
        for f in range(_W):
            @pl.loop(0, _RPT // 16)
            def _tp(k, _f=f):
                rows = k * 16 + lane
                cols = jnp.full((16,), _f, jnp.int32)
                tbuf[_f, pl.ds(k * 16, 16)] = plsc.load_gather(
                    stage, [rows, cols])
            pltpu.sync_copy(tbuf.at[f],
                            acc_hbm.at[(cid * 2 + t) * _W + f, sid])


@functools.cache
def _sc_graph():
  return pl.kernel(
    _sc_body,
    out_type=(jax.ShapeDtypeStruct((_NC * 2 * _W, _NS, _RPT), jnp.float32),
              jax.ShapeDtypeStruct((_EP,), jnp.float32)),
    mesh=plsc.VectorSubcoreMesh(core_axis_name="c", subcore_axis_name="s",
                                num_cores=_NC, num_subcores=_NS),
    scratch_types=[
        pltpu.VMEM((_EPT, _W), jnp.float32),
        pltpu.VMEM((_CPT, _CH), jnp.int32),
        pltpu.VMEM((_GRP, _CH), jnp.int32),
        pltpu.VMEM((_GRP, _CH), jnp.int32),
        pltpu.VMEM((_W, 16), jnp.float32),
        pltpu.VMEM((_EPT,), jnp.float32),
        pltpu.VMEM((_W, _RPT), jnp.float32),
        pltpu.VMEM_SHARED((_NPAD, _W), jnp.float32),
        pltpu.VMEM_SHARED((_NPAD, _W), jnp.float32),
        pltpu.VMEM((_RPT, _W), jnp.float32),
        pltpu.SemaphoreType.DMA,
    ],
    compiler_params=pltpu.CompilerParams(use_tc_tiling_on_sc=False,
                                         needs_layout_passes=False),
  )


def _node_body(nodes_ref, accT_ref, g_ref, Wn_ref, bn_ref, We_ref, be_ref,
               W1a_ref, W1s_ref, W1r_ref, W1g_ref, b1_ref,
               W2_ref, b2_ref, wd_ref, bd_ref, out_ref):
    f32 = jnp.float32
    W1a = W1a_ref[...]
    zero3 = jnp.zeros((3, 10), f32)
    A = jnp.dot(Wn_ref[...], W1a, preferred_element_type=f32)
    Bs = jnp.concatenate(
        [jnp.dot(We_ref[...], W1s_ref[...], preferred_element_type=f32),
         jnp.dot(be_ref[...], W1s_ref[...], preferred_element_type=f32),
         zero3], axis=0)
    Br = jnp.concatenate(
        [jnp.dot(We_ref[...], W1r_ref[...], preferred_element_type=f32),
         jnp.dot(be_ref[...], W1r_ref[...], preferred_element_type=f32),
         zero3], axis=0)
    const = (jnp.dot(bn_ref[...], W1a, preferred_element_type=f32)
             + jnp.dot(g_ref[...], W1g_ref[...], preferred_element_type=f32)
             + b1_ref[...])
    sT = accT_ref[0, 0] + accT_ref[1, 0]
    rT = accT_ref[0, 1] + accT_ref[1, 1]
    dn = (((0,), (0,)), ((), ()))
    segs = lax.dot_general(sT, Bs, dimension_numbers=dn, preferred_element_type=f32)
    segr = lax.dot_general(rT, Br, dimension_numbers=dn, preferred_element_type=f32)
    h = (jnp.dot(nodes_ref[...], A, preferred_element_type=f32)
         + segs[:N] + segr[:N] + const)
    h = jnp.maximum(h, 0.0)
    w2d = jnp.dot(W2_ref[...], wd_ref[...], preferred_element_type=f32)
    cout = jnp.dot(b2_ref[...], wd_ref[...], preferred_element_type=f32) + bd_ref[...]
    out_ref[...] = jnp.dot(h, w2d, preferred_element_type=f32) + cout


def kernel(nodes, edges, senders, receivers, globals_,
           enc_node_W, enc_node_b, enc_edge_W, enc_edge_b,
           mlp_W1, mlp_b1, mlp_W2, mlp_b2,
           dec_node_W, dec_node_b, dec_edge_W, dec_edge_b):
    f32 = jnp.float32
    edges = edges.astype(f32)
    edges8 = jnp.concatenate(
        [edges, jnp.ones((E, 1), f32), jnp.zeros((E, 3), f32)], axis=1)
    edges8 = jnp.pad(edges8, ((0, _EP - E), (0, 0)))
    ew = edges8.reshape(_NW, _EPT, _W)
    pk = ((receivers.astype(jnp.int32) << 16) | senders.astype(jnp.int32))
    pk = jnp.pad(pk, (0, _EP - E)).reshape(-1, 128)
    zeros = jnp.zeros((_RPT, _W), f32)
    v4 = jnp.dot(enc_edge_W, dec_edge_W)[:, 0]
    ebias = jnp.dot(enc_edge_b, dec_edge_W)[0] + dec_edge_b[0]
    vrep = jnp.zeros((_W, 16), f32)
    vrep = vrep.at[0:4].set(jnp.broadcast_to(v4[:, None], (4, 16)))
    vrep = vrep.at[4].set(jnp.broadcast_to(ebias, (16,)))

    acc, eout = _sc_graph()(ew, pk, zeros, vrep)
    accT = acc.reshape(_NC, 2, _W, _NPAD)
    edges_out = eout[:E].reshape(E, 1)

    bn = enc_node_b.reshape(1, -1)
    be = enc_edge_b.reshape(1, -1)
    b1 = mlp_b1.reshape(1, -1)
    b2 = mlp_b2.reshape(1, -1)
    bd = dec_node_b.reshape(1, 1)
    W1a, W1s, W1r, W1g = (mlp_W1[0:10], mlp_W1[10:20], mlp_W1[20:30],
                          mlp_W1[30:34])

    nodes_out = pl.pallas_call(
        _node_body,
        out_shape=jax.ShapeDtypeStruct((N, 1), f32),
    )(nodes, accT, globals_, enc_node_W, bn, enc_edge_W, be,
      W1a, W1s, W1r, W1g, b1, mlp_W2, b2, dec_node_W, bd)

    return nodes_out, edges_out, globals_


# speedup vs baseline: 1.5299x; 1.5299x over previous
import functools

import jax
import jax.numpy as jnp
from jax import lax
from jax.experimental import pallas as pl
from jax.experimental.pallas import tpu as pltpu
from jax.experimental.pallas import tpu_sc as plsc

N = 10000
E = 320000

_NC = 2
_NS = 16
_NW = _NC * _NS
_CH = 128
_NCH = E // _CH
_CPT = _NCH // _NW
_EPT = _CPT * _CH
_XCH = _NCH - _CPT * _NW
_GRP = 5
_NPAD = 10240
_RPT = _NPAD // _NS

_W = 8


def _sc_body(edges_hbm, pidx_hbm, zeros_hbm, vrep_hbm,
             acc_hbm, eout_hbm,
             ebuf, pidx, sb, rb, vbuf, obuf, tbuf, acc_s, acc_r, stage, sem):
    cid = lax.axis_index("c")
    sid = lax.axis_index("s")
    wid = cid * _NS + sid
    extra = wid < _XCH
    pltpu.sync_copy(zeros_hbm, stage)
    pltpu.sync_copy(stage, acc_s.at[pl.ds(sid * _RPT, _RPT)])
    pltpu.sync_copy(stage, acc_r.at[pl.ds(sid * _RPT, _RPT)])
    plsc.subcore_barrier()
    pltpu.sync_copy(edges_hbm.at[pl.ds(wid * _EPT, _EPT)],
                    ebuf.at[pl.ds(0, _EPT)])
    pltpu.sync_copy(pidx_hbm.at[pl.ds(wid * _CPT, _CPT)],
                    pidx.at[pl.ds(0, _CPT)])
    pltpu.sync_copy(vrep_hbm, vbuf)

    @pl.when(extra)
    def _():
        xrow = (_CPT * _NW + wid) * _CH
        pltpu.sync_copy(edges_hbm.at[pl.ds(xrow, _CH)],
                        ebuf.at[pl.ds(_EPT, _CH)])
        pltpu.sync_copy(pidx_hbm.at[pl.ds(_CPT * _NW + wid, 1)],
                        pidx.at[pl.ds(_CPT, 1)])

    lane = lax.iota(jnp.int32, 16)

    def do_group(j0, nch):
        for t in range(nch):
            for c in range(_CH // 16):
                p = pidx[j0 + t, pl.ds(c * 16, 16)]
                sb[t, pl.ds(c * 16, 16)] = lax.bitwise_and(p, 0xFFFF)
                rb[t, pl.ds(c * 16, 16)] = lax.shift_right_logical(p, 16)
        descs = []
        for t in range(nch):
            src = ebuf.at[pl.ds((j0 + t) * _CH, _CH)]
            descs.append(
                pltpu.async_copy(src, acc_s.at[sb.at[t]], sem, add=True))
            descs.append(
                pltpu.async_copy(src, acc_r.at[rb.at[t]], sem, add=True))

        @pl.loop(0, nch * _CH // 16)
        def _dec(k):
            base = j0 * _CH + k * 16
            rows = base + lane
            r16 = vbuf[4]
            for f in range(4):
                cols = jnp.full((16,), f, jnp.int32)
                r16 = r16 + plsc.load_gather(ebuf, [rows, cols]) * vbuf[f]
            obuf[pl.ds(base, 16)] = r16

        for d in descs:
            d.wait()

    @pl.loop(0, 15)
    def _group(g):
        do_group(g * _GRP, _GRP)

    do_group(15 * _GRP, _CPT - 15 * _GRP)

    @pl.when(extra)
    def _():
        do_group(_CPT, 1)

    pltpu.sync_copy(obuf.at[pl.ds(0, _EPT)],
                    eout_hbm.at[pl.ds(wid * _EPT, _EPT)])

    @pl.when(extra)
    def _():
        xrow = (_CPT * _NW + wid) * _CH
        pltpu.sync_copy(obuf.at[pl.ds(_EPT, _CH)],
                        eout_hbm.at[pl.ds(xrow, _CH)])

    plsc.subcore_barrier()
    for t, acc in ((0, acc_s), (1, acc_r)):
        pltpu.sync_copy(acc.at[pl.ds(sid * _RPT, _RPT)], stage)
        for f in range(_W):
            @pl.loop(0, _RPT // 16)
            def _tp(k, _f=f):
                rows = k * 16 + lane
                cols = jnp.full((16,), _f, jnp.int32)
                tbuf[_f, pl.ds(k * 16, 16)] = plsc.load_gather(
                    stage, [rows, cols])
            pltpu.sync_copy(tbuf.at[f],
                            acc_hbm.at[(cid * 2 + t) * _W + f, sid])


@functools.cache
def _sc_graph():
  return pl.kernel(
    _sc_body,
    out_type=(jax.ShapeDtypeStruct((_NC * 2 * _W, _NS, _RPT), jnp.float32),
              jax.ShapeDtypeStruct((E,), jnp.float32)),
    mesh=plsc.VectorSubcoreMesh(core_axis_name="c", subcore_axis_name="s",
                                num_cores=_NC, num_subcores=_NS),
    scratch_types=[
        pltpu.VMEM((_EPT + _CH, _W), jnp.float32),
        pltpu.VMEM((_CPT + 1, _CH), jnp.int32),
        pltpu.VMEM((_GRP, _CH), jnp.int32),
        pltpu.VMEM((_GRP, _CH), jnp.int32),
        pltpu.VMEM((_W, 16), jnp.float32),
        pltpu.VMEM((_EPT + _CH,), jnp.float32),
        pltpu.VMEM((_W, _RPT), jnp.float32),
        pltpu.VMEM_SHARED((_NPAD, _W), jnp.float32),
        pltpu.VMEM_SHARED((_NPAD, _W), jnp.float32),
        pltpu.VMEM((_RPT, _W), jnp.float32),
        pltpu.SemaphoreType.DMA,
    ],
    compiler_params=pltpu.CompilerParams(use_tc_tiling_on_sc=False,
                                         needs_layout_passes=False),
  )


def _node_body(nodes_ref, accT_ref, g_ref, Wn_ref, bn_ref, We_ref, be_ref,
               W1a_ref, W1s_ref, W1r_ref, W1g_ref, b1_ref,
               W2_ref, b2_ref, wd_ref, bd_ref, out_ref):
    f32 = jnp.float32
    W1a = W1a_ref[...]
    zero3 = jnp.zeros((3, 10), f32)
    A = jnp.dot(Wn_ref[...], W1a, preferred_element_type=f32)
    Bs = jnp.concatenate(
        [jnp.dot(We_ref[...], W1s_ref[...], preferred_element_type=f32),
         jnp.dot(be_ref[...], W1s_ref[...], preferred_element_type=f32),
         zero3], axis=0)
    Br = jnp.concatenate(
        [jnp.dot(We_ref[...], W1r_ref[...], preferred_element_type=f32),
         jnp.dot(be_ref[...], W1r_ref[...], preferred_element_type=f32),
         zero3], axis=0)
    const = (jnp.dot(bn_ref[...], W1a, preferred_element_type=f32)
             + jnp.dot(g_ref[...], W1g_ref[...], preferred_element_type=f32)
             + b1_ref[...])
    sT = accT_ref[0, 0] + accT_ref[1, 0]
    rT = accT_ref[0, 1] + accT_ref[1, 1]
    dn = (((0,), (0,)), ((), ()))
    segs = lax.dot_general(sT, Bs, dimension_numbers=dn, preferred_element_type=f32)
    segr = lax.dot_general(rT, Br, dimension_numbers=dn, preferred_element_type=f32)
    h = (jnp.dot(nodes_ref[...], A, preferred_element_type=f32)
         + segs[:N] + segr[:N] + const)
    h = jnp.maximum(h, 0.0)
    w2d = jnp.dot(W2_ref[...], wd_ref[...], preferred_element_type=f32)
    cout = jnp.dot(b2_ref[...], wd_ref[...], preferred_element_type=f32) + bd_ref[...]
    out_ref[...] = jnp.dot(h, w2d, preferred_element_type=f32) + cout


def kernel(nodes, edges, senders, receivers, globals_,
           enc_node_W, enc_node_b, enc_edge_W, enc_edge_b,
           mlp_W1, mlp_b1, mlp_W2, mlp_b2,
           dec_node_W, dec_node_b, dec_edge_W, dec_edge_b):
    f32 = jnp.float32
    edges = edges.astype(f32)
    edges8 = jnp.concatenate(
        [edges, jnp.ones((E, 1), f32), jnp.zeros((E, 3), f32)], axis=1)
    pk = ((receivers.astype(jnp.int32) << 16)
          | senders.astype(jnp.int32)).reshape(_NCH, _CH)
    zeros = jnp.zeros((_RPT, _W), f32)
    v4 = jnp.dot(enc_edge_W, dec_edge_W)[:, 0]
    ebias = jnp.dot(enc_edge_b, dec_edge_W)[0] + dec_edge_b[0]
    vrep = jnp.zeros((_W, 16), f32)
    vrep = vrep.at[0:4].set(jnp.broadcast_to(v4[:, None], (4, 16)))
    vrep = vrep.at[4].set(jnp.broadcast_to(ebias, (16,)))

    acc, eout = _sc_graph()(edges8, pk, zeros, vrep)
    accT = acc.reshape(_NC, 2, _W, _NPAD)
    edges_out = eout.reshape(E, 1)

    bn = enc_node_b.reshape(1, -1)
    be = enc_edge_b.reshape(1, -1)
    b1 = mlp_b1.reshape(1, -1)
    b2 = mlp_b2.reshape(1, -1)
    bd = dec_node_b.reshape(1, 1)
    W1a, W1s, W1r, W1g = (mlp_W1[0:10], mlp_W1[10:20], mlp_W1[20:30],
                          mlp_W1[30:34])

    nodes_out = pl.pallas_call(
        _node_body,
        out_shape=jax.ShapeDtypeStruct((N, 1), f32),
    )(nodes, accT, globals_, enc_node_W, bn, enc_edge_W, be,
      W1a, W1s, W1r, W1g, b1, mlp_W2, b2, dec_node_W, bd)

    return nodes_out, edges_out, globals_
